# baseline (speedup 1.0000x reference)
import jax
import jax.numpy as jnp
from jax import lax
from jax.experimental import pallas as pl
from jax.experimental.pallas import tpu as pltpu
from jax.experimental.pallas import tpu_sc as plsc

N = 10000
NR = 10240
D = 128
E = 320000
NC = 2
NS = 16
NW = NC * NS
CHUNK = 80
NCHUNK = 128
SEG = 16
NSEG = NCHUNK // SEG
EW = NCHUNK * CHUNK
EPAD = NW * EW
ROWS_PT = NR // NS

BM = 1024
GRID = NR // BM


def _mesh():
    return plsc.VectorSubcoreMesh(
        core_axis_name="c", subcore_axis_name="s",
        num_cores=NC, num_subcores=NS)


def _deg_body(dst_hbm, ones_hbm, zeros_hbm, deg_out, dst_v, ones_v, shared_deg):
    cid = lax.axis_index("c")
    sid = lax.axis_index("s")
    wid = cid * NS + sid
    pltpu.sync_copy(dst_hbm.at[wid], dst_v)
    pltpu.sync_copy(ones_hbm, ones_v)
    pltpu.sync_copy(zeros_hbm.at[pl.ds(0, 1), pl.ds(sid * ROWS_PT, ROWS_PT)],
                    shared_deg.at[pl.ds(0, 1), pl.ds(sid * ROWS_PT, ROWS_PT)])
    plsc.subcore_barrier()

    def body(j, carry):
        pltpu.sync_copy(ones_v, shared_deg.at[dst_v.at[j]], add=True)
        return carry

    lax.fori_loop(0, NCHUNK, body, 0)
    plsc.subcore_barrier()
    pltpu.sync_copy(shared_deg.at[pl.ds(0, 1), pl.ds(sid * ROWS_PT, ROWS_PT)],
                    deg_out.at[cid, pl.ds(0, 1), pl.ds(sid * ROWS_PT, ROWS_PT)])


def _sc_deg(dst4):
    f = pl.kernel(
        _deg_body,
        out_type=jax.ShapeDtypeStruct((NC, 1, NR), jnp.float32),
        mesh=_mesh(),
        scratch_types=[
            pltpu.VMEM((NCHUNK, 1, CHUNK), jnp.int32),
            pltpu.VMEM((1, CHUNK), jnp.float32),
            pltpu.VMEM_SHARED((1, NR), jnp.float32),
        ],
    )
    return f(dst4, jnp.ones((1, CHUNK), jnp.float32),
             jnp.zeros((1, NR), jnp.float32))


def _scatter_body(g_hbm, src_hbm, dst_hbm, agg_out,
                  src_v, dst_v, rows0, rows1, shared_g, sem0, sem1):
    cid = lax.axis_index("c")
    sid = lax.axis_index("s")
    wid = cid * NS + sid
    pltpu.sync_copy(g_hbm.at[pl.ds(sid * ROWS_PT, ROWS_PT)],
                    shared_g.at[pl.ds(sid * ROWS_PT, ROWS_PT)])
    plsc.subcore_barrier()

    for s in range(NSEG):
        pltpu.sync_copy(src_hbm.at[wid, pl.ds(s * SEG, SEG)], src_v)
        pltpu.sync_copy(dst_hbm.at[wid, pl.ds(s * SEG, SEG)], dst_v)
        def body(j, carry):
            pltpu.sync_copy(g_hbm.at[src_v.at[j]], rows0)
            pltpu.sync_copy(rows0, shared_g.at[dst_v.at[j]], add=True)
            return carry

        lax.fori_loop(0, SEG, body, 0)
    plsc.subcore_barrier()
    pltpu.sync_copy(shared_g.at[pl.ds(sid * ROWS_PT, ROWS_PT)],
                    agg_out.at[cid, pl.ds(sid * ROWS_PT, ROWS_PT)])


def _sc_scatter(g, src4, dst4):
    f = pl.kernel(
        _scatter_body,
        out_type=jax.ShapeDtypeStruct((NC, NR, D), jnp.float32),
        mesh=_mesh(),
        scratch_types=[
            pltpu.VMEM((SEG, CHUNK), jnp.int32),
            pltpu.VMEM((SEG, CHUNK), jnp.int32),
            pltpu.VMEM((CHUNK, D), jnp.float32),
            pltpu.VMEM((CHUNK, D), jnp.float32),
            pltpu.VMEM_SHARED((NR, D), jnp.float32),
            pltpu.SemaphoreType.DMA,
            pltpu.SemaphoreType.DMA,
        ],
    )
    return f(g, src4, dst4)


def _k1_body(x_ref, w_ref, deg_ref, g_ref):
    dinv = lax.rsqrt(deg_ref[0, :, :] + deg_ref[1, :, :] + 1.0)
    g_ref[...] = jnp.dot(x_ref[...], w_ref[...],
                         preferred_element_type=jnp.float32) * dinv


def _k1(x, W1, degp3):
    return pl.pallas_call(
        _k1_body,
        grid=(GRID,),
        in_specs=[
            pl.BlockSpec((BM, D), lambda i: (i, 0)),
            pl.BlockSpec((D, D), lambda i: (0, 0)),
            pl.BlockSpec((NC, BM, 1), lambda i: (0, i, 0)),
        ],
        out_specs=pl.BlockSpec((BM, D), lambda i: (i, 0)),
        out_shape=jax.ShapeDtypeStruct((NR, D), jnp.float32),
    )(x, W1, degp3)


def _k2_body(agg_ref, g_ref, deg_ref, w_ref, b_ref, out_ref):
    dinv = lax.rsqrt(deg_ref[0, :, :] + deg_ref[1, :, :] + 1.0)
    h = dinv * (agg_ref[0, :, :] + agg_ref[1, :, :] - g_ref[...]) + b_ref[...]
    h = jnp.maximum(h, 0.0)
    out_ref[...] = jnp.dot(h, w_ref[...],
                           preferred_element_type=jnp.float32) * dinv


def _k2(aggp, g1, degp3, W2, b1r):
    return pl.pallas_call(
        _k2_body,
        grid=(GRID,),
        in_specs=[
            pl.BlockSpec((NC, BM, D), lambda i: (0, i, 0)),
            pl.BlockSpec((BM, D), lambda i: (i, 0)),
            pl.BlockSpec((NC, BM, 1), lambda i: (0, i, 0)),
            pl.BlockSpec((D, D), lambda i: (0, 0)),
            pl.BlockSpec((1, D), lambda i: (0, 0)),
        ],
        out_specs=pl.BlockSpec((BM, D), lambda i: (i, 0)),
        out_shape=jax.ShapeDtypeStruct((NR, D), jnp.float32),
    )(aggp, g1, degp3, W2, b1r)


def _k3_body(agg_ref, g_ref, deg_ref, b_ref, wfc_ref, bfc_ref, out_ref):
    dinv = lax.rsqrt(deg_ref[0, :, :] + deg_ref[1, :, :] + 1.0)
    h = dinv * (agg_ref[0, :, :] + agg_ref[1, :, :] - g_ref[...]) + b_ref[...]
    h = jnp.maximum(h, 0.0)
    out_ref[...] = jnp.dot(h, wfc_ref[...],
                           preferred_element_type=jnp.float32) + bfc_ref[...]


def _k3(aggp, g2, degp3, b2r, Wfc, bfcr):
    return pl.pallas_call(
        _k3_body,
        grid=(GRID,),
        in_specs=[
            pl.BlockSpec((NC, BM, D), lambda i: (0, i, 0)),
            pl.BlockSpec((BM, D), lambda i: (i, 0)),
            pl.BlockSpec((NC, BM, 1), lambda i: (0, i, 0)),
            pl.BlockSpec((1, D), lambda i: (0, 0)),
            pl.BlockSpec((D, 1), lambda i: (0, 0)),
            pl.BlockSpec((1, 1), lambda i: (0, 0)),
        ],
        out_specs=pl.BlockSpec((BM, 1), lambda i: (i, 0)),
        out_shape=jax.ShapeDtypeStruct((NR, 1), jnp.float32),
    )(aggp, g2, degp3, b2r, Wfc, bfcr)


def kernel(x, edge_index, W1, b1, W2, b2, Wfc, bfc):
    pad = jnp.full((2, EPAD - E), N, jnp.int32)
    ei = jnp.concatenate([edge_index, pad], axis=1)
    src3 = ei[0].reshape(NW, NCHUNK, CHUNK)
    dst4 = ei[1].reshape(NW, NCHUNK, 1, CHUNK)
    dst3 = ei[1].reshape(NW, NCHUNK, CHUNK)

    degp3 = _sc_deg(dst4).reshape(NC, NR, 1)

    g1 = _k1(x, W1, degp3)
    agg1 = _sc_scatter(g1, src3, dst3)
    g2 = _k2(agg1, g1, degp3, W2, b1.reshape(1, D))
    agg2 = _sc_scatter(g2, src3, dst3)
    out = _k3(agg2, g2, degp3, b2.reshape(1, D), Wfc, bfc.reshape(1, 1))
    return out[:N, 0]


# speedup vs baseline: 3.5829x; 3.5829x over previous
import jax
import jax.numpy as jnp
from jax import lax
from jax.experimental import pallas as pl
from jax.experimental.pallas import tpu as pltpu
from jax.experimental.pallas import tpu_sc as plsc

N = 10000
NR = 10240
D = 128
E = 320000
NC = 2
NS = 16
NW = NC * NS
CHUNK = 128
NCHUNK = 80
SEG = 16
NSEG = NCHUNK // SEG
EW = NCHUNK * CHUNK
EPAD = NW * EW
ROWS_PT = NR // NS

BM = 1024
GRID = NR // BM


def _mesh():
    return plsc.VectorSubcoreMesh(
        core_axis_name="c", subcore_axis_name="s",
        num_cores=NC, num_subcores=NS)


def _deg_body(dst_hbm, ones_hbm, zeros_hbm, deg_out, dst_v, ones_v, shared_deg):
    cid = lax.axis_index("c")
    sid = lax.axis_index("s")
    wid = cid * NS + sid
    pltpu.sync_copy(dst_hbm.at[wid], dst_v)
    pltpu.sync_copy(ones_hbm, ones_v)
    pltpu.sync_copy(zeros_hbm.at[pl.ds(0, 1), pl.ds(sid * ROWS_PT, ROWS_PT)],
                    shared_deg.at[pl.ds(0, 1), pl.ds(sid * ROWS_PT, ROWS_PT)])
    plsc.subcore_barrier()

    def body(j, carry):
        pltpu.sync_copy(ones_v, shared_deg.at[dst_v.at[j]], add=True)
        return carry

    lax.fori_loop(0, NCHUNK, body, 0)
    plsc.subcore_barrier()
    pltpu.sync_copy(shared_deg.at[pl.ds(0, 1), pl.ds(sid * ROWS_PT, ROWS_PT)],
                    deg_out.at[cid, pl.ds(0, 1), pl.ds(sid * ROWS_PT, ROWS_PT)])


def _sc_deg(dst4):
    f = pl.kernel(
        _deg_body,
        out_type=jax.ShapeDtypeStruct((NC, 1, NR), jnp.float32),
        mesh=_mesh(),
        scratch_types=[
            pltpu.VMEM((NCHUNK, 1, CHUNK), jnp.int32),
            pltpu.VMEM((1, CHUNK), jnp.float32),
            pltpu.VMEM_SHARED((1, NR), jnp.float32),
        ],
    )
    return f(dst4, jnp.ones((1, CHUNK), jnp.float32),
             jnp.zeros((1, NR), jnp.float32))


def _scatter_body(g_hbm, src_hbm, dst_hbm, agg_out,
                  src_v, dst_v, rows0, rows1, shared_g, sem0, sem1):
    cid = lax.axis_index("c")
    sid = lax.axis_index("s")
    wid = cid * NS + sid
    pltpu.sync_copy(g_hbm.at[pl.ds(sid * ROWS_PT, ROWS_PT)],
                    shared_g.at[pl.ds(sid * ROWS_PT, ROWS_PT)])
    plsc.subcore_barrier()

    for s in range(NSEG):
        pltpu.sync_copy(src_hbm.at[wid, pl.ds(s * SEG, SEG)], src_v)
        pltpu.sync_copy(dst_hbm.at[wid, pl.ds(s * SEG, SEG)], dst_v)
        pltpu.async_copy(g_hbm.at[src_v.at[0]], rows0, sem0)
        pltpu.async_copy(g_hbm.at[src_v.at[1]], rows1, sem1)

        def body(i, carry):
            j = 2 * i
            pltpu.make_async_copy(g_hbm.at[src_v.at[j]], rows0, sem0).wait()
            pltpu.sync_copy(rows0, shared_g.at[dst_v.at[j]], add=True)
            pltpu.async_copy(g_hbm.at[src_v.at[j + 2]], rows0, sem0)
            pltpu.make_async_copy(g_hbm.at[src_v.at[j + 1]], rows1, sem1).wait()
            pltpu.sync_copy(rows1, shared_g.at[dst_v.at[j + 1]], add=True)
            pltpu.async_copy(g_hbm.at[src_v.at[j + 3]], rows1, sem1)
            return carry

        lax.fori_loop(0, SEG // 2 - 1, body, 0)
        j = SEG - 2
        pltpu.make_async_copy(g_hbm.at[src_v.at[j]], rows0, sem0).wait()
        pltpu.sync_copy(rows0, shared_g.at[dst_v.at[j]], add=True)
        pltpu.make_async_copy(g_hbm.at[src_v.at[j + 1]], rows1, sem1).wait()
        pltpu.sync_copy(rows1, shared_g.at[dst_v.at[j + 1]], add=True)
    plsc.subcore_barrier()
    pltpu.sync_copy(shared_g.at[pl.ds(sid * ROWS_PT, ROWS_PT)],
                    agg_out.at[cid, pl.ds(sid * ROWS_PT, ROWS_PT)])


def _sc_scatter(g, src4, dst4):
    f = pl.kernel(
        _scatter_body,
        out_type=jax.ShapeDtypeStruct((NC, NR, D), jnp.float32),
        mesh=_mesh(),
        scratch_types=[
            pltpu.VMEM((SEG, CHUNK), jnp.int32),
            pltpu.VMEM((SEG, CHUNK), jnp.int32),
            pltpu.VMEM((CHUNK, D), jnp.float32),
            pltpu.VMEM((CHUNK, D), jnp.float32),
            pltpu.VMEM_SHARED((NR, D), jnp.float32),
            pltpu.SemaphoreType.DMA,
            pltpu.SemaphoreType.DMA,
        ],
    )
    return f(g, src4, dst4)


def _k1_body(x_ref, w_ref, deg_ref, g_ref):
    dinv = lax.rsqrt(deg_ref[0, :, :] + deg_ref[1, :, :] + 1.0)
    g_ref[...] = jnp.dot(x_ref[...], w_ref[...],
                         preferred_element_type=jnp.float32) * dinv


def _k1(x, W1, degp3):
    return pl.pallas_call(
        _k1_body,
        grid=(GRID,),
        in_specs=[
            pl.BlockSpec((BM, D), lambda i: (i, 0)),
            pl.BlockSpec((D, D), lambda i: (0, 0)),
            pl.BlockSpec((NC, BM, 1), lambda i: (0, i, 0)),
        ],
        out_specs=pl.BlockSpec((BM, D), lambda i: (i, 0)),
        out_shape=jax.ShapeDtypeStruct((NR, D), jnp.float32),
    )(x, W1, degp3)


def _k2_body(agg_ref, g_ref, deg_ref, w_ref, b_ref, out_ref):
    dinv = lax.rsqrt(deg_ref[0, :, :] + deg_ref[1, :, :] + 1.0)
    h = dinv * (agg_ref[0, :, :] + agg_ref[1, :, :] - g_ref[...]) + b_ref[...]
    h = jnp.maximum(h, 0.0)
    out_ref[...] = jnp.dot(h, w_ref[...],
                           preferred_element_type=jnp.float32) * dinv


def _k2(aggp, g1, degp3, W2, b1r):
    return pl.pallas_call(
        _k2_body,
        grid=(GRID,),
        in_specs=[
            pl.BlockSpec((NC, BM, D), lambda i: (0, i, 0)),
            pl.BlockSpec((BM, D), lambda i: (i, 0)),
            pl.BlockSpec((NC, BM, 1), lambda i: (0, i, 0)),
            pl.BlockSpec((D, D), lambda i: (0, 0)),
            pl.BlockSpec((1, D), lambda i: (0, 0)),
        ],
        out_specs=pl.BlockSpec((BM, D), lambda i: (i, 0)),
        out_shape=jax.ShapeDtypeStruct((NR, D), jnp.float32),
    )(aggp, g1, degp3, W2, b1r)


def _k3_body(agg_ref, g_ref, deg_ref, b_ref, wfc_ref, bfc_ref, out_ref):
    dinv = lax.rsqrt(deg_ref[0, :, :] + deg_ref[1, :, :] + 1.0)
    h = dinv * (agg_ref[0, :, :] + agg_ref[1, :, :] - g_ref[...]) + b_ref[...]
    h = jnp.maximum(h, 0.0)
    out_ref[...] = jnp.dot(h, wfc_ref[...],
                           preferred_element_type=jnp.float32) + bfc_ref[...]


def _k3(aggp, g2, degp3, b2r, Wfc, bfcr):
    return pl.pallas_call(
        _k3_body,
        grid=(GRID,),
        in_specs=[
            pl.BlockSpec((NC, BM, D), lambda i: (0, i, 0)),
            pl.BlockSpec((BM, D), lambda i: (i, 0)),
            pl.BlockSpec((NC, BM, 1), lambda i: (0, i, 0)),
            pl.BlockSpec((1, D), lambda i: (0, 0)),
            pl.BlockSpec((D, 1), lambda i: (0, 0)),
            pl.BlockSpec((1, 1), lambda i: (0, 0)),
        ],
        out_specs=pl.BlockSpec((BM, 1), lambda i: (i, 0)),
        out_shape=jax.ShapeDtypeStruct((NR, 1), jnp.float32),
    )(aggp, g2, degp3, b2r, Wfc, bfcr)


def kernel(x, edge_index, W1, b1, W2, b2, Wfc, bfc):
    pad_row = N + (jnp.arange(EPAD - E, dtype=jnp.int32) % (NR - N))
    pad = jnp.stack([pad_row, pad_row])
    ei = jnp.concatenate([edge_index, pad], axis=1)
    src3 = ei[0].reshape(NW, NCHUNK, CHUNK)
    dst4 = ei[1].reshape(NW, NCHUNK, 1, CHUNK)
    dst3 = ei[1].reshape(NW, NCHUNK, CHUNK)

    degp3 = _sc_deg(dst4).reshape(NC, NR, 1)

    g1 = _k1(x, W1, degp3)
    agg1 = _sc_scatter(g1, src3, dst3)
    g2 = _k2(agg1, g1, degp3, W2, b1.reshape(1, D))
    agg2 = _sc_scatter(g2, src3, dst3)
    out = _k3(agg2, g2, degp3, b2.reshape(1, D), Wfc, bfc.reshape(1, 1))
    return out[:N, 0]
